# baseline (speedup 1.0000x reference)
import functools

import jax
import jax.numpy as jnp
from jax import lax
from jax.experimental import pallas as pl
from jax.experimental.pallas import tpu as pltpu
from jax.experimental.pallas import tpu_sc as plsc

BATCH = 16384
NROWS = 1000000
EMBED_DIM = 64
LANES = 16
IDX_CHUNK = 128


def _mf_body(users_ref, items_ref, uet_ref, iet_ref, ub_ref, ib_ref, gb_ref,
             out_ref, uidx_v, iidx_v, urows_v, irows_v,
             ub_v, ib_v, scores_v, gb_v, sem_bias, sem_emb, *,
             b_per_w, num_cores):
    wid = lax.axis_index("s") * num_cores + lax.axis_index("c")
    base = wid * b_per_w
    nchunk = b_per_w // IDX_CHUNK

    pltpu.sync_copy(users_ref.at[pl.ds(base, b_per_w)], uidx_v)
    pltpu.sync_copy(items_ref.at[pl.ds(base, b_per_w)], iidx_v)
    pltpu.sync_copy(gb_ref, gb_v.at[pl.ds(0, 1)])

    bias_handles = []
    for j in range(nchunk):
        sl = pl.ds(j * IDX_CHUNK, IDX_CHUNK)
        bias_handles.append(pltpu.async_copy(
            ub_ref.at[uidx_v.at[sl]], ub_v.at[sl], sem_bias))
        bias_handles.append(pltpu.async_copy(
            ib_ref.at[iidx_v.at[sl]], ib_v.at[sl], sem_bias))

    for c in range(EMBED_DIM):
        for j in range(nchunk):
            sl = pl.ds(j * IDX_CHUNK, IDX_CHUNK)
            pltpu.async_copy(uet_ref.at[c].at[uidx_v.at[sl]],
                             urows_v.at[c, sl], sem_emb)
            pltpu.async_copy(iet_ref.at[c].at[iidx_v.at[sl]],
                             irows_v.at[c, sl], sem_emb)

    pltpu.make_async_copy(uet_ref.at[:, pl.ds(0, b_per_w)], urows_v,
                          sem_emb).wait()
    pltpu.make_async_copy(iet_ref.at[:, pl.ds(0, b_per_w)], irows_v,
                          sem_emb).wait()
    for h in bias_handles:
        h.wait()

    gb = gb_v[pl.ds(0, LANES)][0]

    def group(g, _):
        sl = pl.ds(g * LANES, LANES)
        acc = jnp.zeros((LANES,), jnp.float32)
        for c in range(EMBED_DIM):
            acc = acc + urows_v[c, sl] * irows_v[c, sl]
        scores_v[sl] = acc + ub_v[sl] + ib_v[sl] + gb
        return 0

    lax.fori_loop(0, b_per_w // LANES, group, 0)

    pltpu.sync_copy(scores_v, out_ref.at[pl.ds(base, b_per_w)])


def kernel(users, items, user_embedding, item_embedding, user_bias,
           item_bias, global_bias):
    info = plsc.get_sparse_core_info()
    num_workers = info.num_cores * info.num_subcores
    b_per_w = BATCH // num_workers

    mesh = plsc.VectorSubcoreMesh(core_axis_name="c", subcore_axis_name="s")
    k = pl.kernel(
        functools.partial(_mf_body, b_per_w=b_per_w,
                          num_cores=info.num_cores),
        mesh=mesh,
        compiler_params=pltpu.CompilerParams(use_tc_tiling_on_sc=False),
        out_type=jax.ShapeDtypeStruct((BATCH,), jnp.float32),
        scratch_types=[
            pltpu.VMEM((b_per_w,), jnp.int32),
            pltpu.VMEM((b_per_w,), jnp.int32),
            pltpu.VMEM((EMBED_DIM, b_per_w), jnp.float32),
            pltpu.VMEM((EMBED_DIM, b_per_w), jnp.float32),
            pltpu.VMEM((b_per_w,), jnp.float32),
            pltpu.VMEM((b_per_w,), jnp.float32),
            pltpu.VMEM((b_per_w,), jnp.float32),
            pltpu.VMEM((LANES,), jnp.float32),
            pltpu.SemaphoreType.DMA,
            pltpu.SemaphoreType.DMA,
        ],
    )
    return k(users.astype(jnp.int32), items.astype(jnp.int32),
             user_embedding.T, item_embedding.T,
             user_bias.reshape(user_bias.shape[0]),
             item_bias.reshape(item_bias.shape[0]),
             global_bias)


# speedup vs baseline: 18.8378x; 18.8378x over previous
import functools

import jax
import jax.numpy as jnp
from jax import lax
from jax.experimental import pallas as pl
from jax.experimental.pallas import tpu as pltpu
from jax.experimental.pallas import tpu_sc as plsc

BATCH = 16384
NROWS = 1000000
EMBED_DIM = 64
LANES = 16
IDX_CHUNK = 128
RING = 4
SLAB_BYTES = 2 * 8 * 8 * 128 * 4


def _mf_body(users_ref, items_ref, ue3_ref, ie3_ref, ub_ref, ib_ref, gb_ref,
             out_ref, uidx_v, iidx_v, ublk, iblk, ub_v, ib_v, scores_v, gb_v,
             sem_bias, sem0, sem1, sem2, sem3, *, b_per_w, num_cores):
    wid = lax.axis_index("s") * num_cores + lax.axis_index("c")
    base = wid * b_per_w
    nchunk = b_per_w // IDX_CHUNK
    sems = [sem0, sem1, sem2, sem3]

    pltpu.sync_copy(users_ref.at[pl.ds(base, b_per_w)], uidx_v)
    pltpu.sync_copy(items_ref.at[pl.ds(base, b_per_w)], iidx_v)
    pltpu.sync_copy(gb_ref, gb_v.at[pl.ds(0, 1)])

    bias_handles = []
    for j in range(nchunk):
        sl = pl.ds(j * IDX_CHUNK, IDX_CHUNK)
        bias_handles.append(pltpu.async_copy(
            ub_ref.at[uidx_v.at[sl]], ub_v.at[sl], sem_bias))
        bias_handles.append(pltpu.async_copy(
            ib_ref.at[iidx_v.at[sl]], ib_v.at[sl], sem_bias))
    for h in bias_handles:
        h.wait()

    gb = gb_v[pl.ds(0, LANES)][0]
    lane = lax.iota(jnp.int32, LANES)
    chiv = [(m * LANES + lane) >> 3 for m in range(EMBED_DIM // LANES)]
    clov = [(m * LANES + lane) & 7 for m in range(EMBED_DIM // LANES)]

    def fire(slot, ru, ri):
        u0 = pl.multiple_of((ru >> 7) * 128, 128)
        i0 = pl.multiple_of((ri >> 7) * 128, 128)
        pltpu.async_copy(ue3_ref.at[:, :, pl.ds(u0, 128)],
                         ublk.at[slot], sems[slot])
        pltpu.async_copy(ie3_ref.at[:, :, pl.ds(i0, 128)],
                         iblk.at[slot], sems[slot])

    def group(g, _):
        sl = pl.ds(g * LANES, LANES)
        ru = uidx_v[sl]
        ri = iidx_v[sl]
        for j in range(min(RING - 1, LANES)):
            fire(j % RING, ru[j], ri[j])
        score_vec = jnp.zeros((LANES,), jnp.float32)
        for j in range(LANES):
            slot = j % RING
            if j + RING - 1 < LANES:
                fire((j + RING - 1) % RING, ru[j + RING - 1], ri[j + RING - 1])
            pltpu.make_async_copy(
                ue3_ref.at[:, :, pl.ds(0, 128)], ublk.at[slot],
                sems[slot]).wait()
            pltpu.make_async_copy(
                ie3_ref.at[:, :, pl.ds(0, 128)], iblk.at[slot],
                sems[slot]).wait()
            rum = lax.broadcast(ru[j] & 127, (LANES,))
            rim = lax.broadcast(ri[j] & 127, (LANES,))
            ks = lax.broadcast(jnp.int32(slot), (LANES,))
            acc = None
            for m in range(EMBED_DIM // LANES):
                u16 = plsc.load_gather(ublk, [ks, chiv[m], clov[m], rum])
                i16 = plsc.load_gather(iblk, [ks, chiv[m], clov[m], rim])
                p = u16 * i16
                acc = p if acc is None else acc + p
            s_val = jnp.sum(acc)
            score_vec = jnp.where(lane == j, s_val, score_vec)
        scores_v[sl] = score_vec + ub_v[sl] + ib_v[sl] + gb
        return 0

    lax.fori_loop(0, b_per_w // LANES, group, 0)

    pltpu.sync_copy(scores_v, out_ref.at[pl.ds(base, b_per_w)])


def kernel(users, items, user_embedding, item_embedding, user_bias,
           item_bias, global_bias):
    info = plsc.get_sparse_core_info()
    num_workers = info.num_cores * info.num_subcores
    b_per_w = BATCH // num_workers

    mesh = plsc.VectorSubcoreMesh(core_axis_name="c", subcore_axis_name="s")
    k = pl.kernel(
        functools.partial(_mf_body, b_per_w=b_per_w,
                          num_cores=info.num_cores),
        mesh=mesh,
        compiler_params=pltpu.CompilerParams(use_tc_tiling_on_sc=True,
                                             needs_layout_passes=False),
        out_type=jax.ShapeDtypeStruct((BATCH,), jnp.float32),
        scratch_types=[
            pltpu.VMEM((b_per_w,), jnp.int32),
            pltpu.VMEM((b_per_w,), jnp.int32),
            pltpu.VMEM((RING, 8, 8, 128), jnp.float32),
            pltpu.VMEM((RING, 8, 8, 128), jnp.float32),
            pltpu.VMEM((b_per_w,), jnp.float32),
            pltpu.VMEM((b_per_w,), jnp.float32),
            pltpu.VMEM((b_per_w,), jnp.float32),
            pltpu.VMEM((LANES,), jnp.float32),
            pltpu.SemaphoreType.DMA,
            pltpu.SemaphoreType.DMA,
            pltpu.SemaphoreType.DMA,
            pltpu.SemaphoreType.DMA,
            pltpu.SemaphoreType.DMA,
        ],
    )
    return k(users.astype(jnp.int32), items.astype(jnp.int32),
             user_embedding.T.reshape(8, 8, NROWS),
             item_embedding.T.reshape(8, 8, NROWS),
             user_bias.reshape(user_bias.shape[0]),
             item_bias.reshape(item_bias.shape[0]),
             global_bias)


# speedup vs baseline: 19.4890x; 1.0346x over previous
import functools

import jax
import jax.numpy as jnp
from jax import lax
from jax.experimental import pallas as pl
from jax.experimental.pallas import tpu as pltpu
from jax.experimental.pallas import tpu_sc as plsc

BATCH = 16384
NROWS = 1000000
EMBED_DIM = 64
LANES = 16
IDX_CHUNK = 128
RING = 6
SLAB_BYTES = 2 * 8 * 8 * 128 * 4


def _mf_body(users_ref, items_ref, ue3_ref, ie3_ref, ub_ref, ib_ref, gb_ref,
             out_ref, uidx_v, iidx_v, ublk, iblk, ub_v, ib_v, scores_v, gb_v,
             sem_bias, sem0, sem1, sem2, sem3, sem4, sem5, *, b_per_w, num_cores):
    wid = lax.axis_index("s") * num_cores + lax.axis_index("c")
    base = wid * b_per_w
    nchunk = b_per_w // IDX_CHUNK
    sems = [sem0, sem1, sem2, sem3, sem4, sem5]

    pltpu.sync_copy(users_ref.at[pl.ds(base, b_per_w)], uidx_v)
    pltpu.sync_copy(items_ref.at[pl.ds(base, b_per_w)], iidx_v)
    pltpu.sync_copy(gb_ref, gb_v.at[pl.ds(0, 1)])

    bias_handles = []
    for j in range(nchunk):
        sl = pl.ds(j * IDX_CHUNK, IDX_CHUNK)
        bias_handles.append(pltpu.async_copy(
            ub_ref.at[uidx_v.at[sl]], ub_v.at[sl], sem_bias))
        bias_handles.append(pltpu.async_copy(
            ib_ref.at[iidx_v.at[sl]], ib_v.at[sl], sem_bias))
    for h in bias_handles:
        h.wait()

    gb = gb_v[pl.ds(0, LANES)][0]
    lane = lax.iota(jnp.int32, LANES)
    chiv = [(m * LANES + lane) >> 3 for m in range(EMBED_DIM // LANES)]
    clov = [(m * LANES + lane) & 7 for m in range(EMBED_DIM // LANES)]

    def fire(slot, ru, ri):
        u0 = pl.multiple_of((ru >> 7) * 128, 128)
        i0 = pl.multiple_of((ri >> 7) * 128, 128)
        pltpu.async_copy(ue3_ref.at[:, :, pl.ds(u0, 128)],
                         ublk.at[slot], sems[slot])
        pltpu.async_copy(ie3_ref.at[:, :, pl.ds(i0, 128)],
                         iblk.at[slot], sems[slot])

    def group(g, _):
        sl = pl.ds(g * LANES, LANES)
        ru = uidx_v[sl]
        ri = iidx_v[sl]
        for j in range(min(RING - 1, LANES)):
            fire(j % RING, ru[j], ri[j])
        score_vec = jnp.zeros((LANES,), jnp.float32)
        for j in range(LANES):
            slot = j % RING
            if j + RING - 1 < LANES:
                fire((j + RING - 1) % RING, ru[j + RING - 1], ri[j + RING - 1])
            pltpu.make_async_copy(
                ue3_ref.at[:, :, pl.ds(0, 128)], ublk.at[slot],
                sems[slot]).wait()
            pltpu.make_async_copy(
                ie3_ref.at[:, :, pl.ds(0, 128)], iblk.at[slot],
                sems[slot]).wait()
            rum = lax.broadcast(ru[j] & 127, (LANES,))
            rim = lax.broadcast(ri[j] & 127, (LANES,))
            ks = lax.broadcast(jnp.int32(slot), (LANES,))
            acc = None
            for m in range(EMBED_DIM // LANES):
                u16 = plsc.load_gather(ublk, [ks, chiv[m], clov[m], rum])
                i16 = plsc.load_gather(iblk, [ks, chiv[m], clov[m], rim])
                p = u16 * i16
                acc = p if acc is None else acc + p
            s_val = jnp.sum(acc)
            score_vec = jnp.where(lane == j, s_val, score_vec)
        scores_v[sl] = score_vec + ub_v[sl] + ib_v[sl] + gb
        return 0

    lax.fori_loop(0, b_per_w // LANES, group, 0)

    pltpu.sync_copy(scores_v, out_ref.at[pl.ds(base, b_per_w)])


def kernel(users, items, user_embedding, item_embedding, user_bias,
           item_bias, global_bias):
    info = plsc.get_sparse_core_info()
    num_workers = info.num_cores * info.num_subcores
    b_per_w = BATCH // num_workers

    mesh = plsc.VectorSubcoreMesh(core_axis_name="c", subcore_axis_name="s")
    k = pl.kernel(
        functools.partial(_mf_body, b_per_w=b_per_w,
                          num_cores=info.num_cores),
        mesh=mesh,
        compiler_params=pltpu.CompilerParams(use_tc_tiling_on_sc=True,
                                             needs_layout_passes=False),
        out_type=jax.ShapeDtypeStruct((BATCH,), jnp.float32),
        scratch_types=[
            pltpu.VMEM((b_per_w,), jnp.int32),
            pltpu.VMEM((b_per_w,), jnp.int32),
            pltpu.VMEM((RING, 8, 8, 128), jnp.float32),
            pltpu.VMEM((RING, 8, 8, 128), jnp.float32),
            pltpu.VMEM((b_per_w,), jnp.float32),
            pltpu.VMEM((b_per_w,), jnp.float32),
            pltpu.VMEM((b_per_w,), jnp.float32),
            pltpu.VMEM((LANES,), jnp.float32),
            pltpu.SemaphoreType.DMA,
            pltpu.SemaphoreType.DMA,
            pltpu.SemaphoreType.DMA,
            pltpu.SemaphoreType.DMA,
            pltpu.SemaphoreType.DMA,
            pltpu.SemaphoreType.DMA,
            pltpu.SemaphoreType.DMA,
        ],
    )
    return k(users.astype(jnp.int32), items.astype(jnp.int32),
             user_embedding.T.reshape(8, 8, NROWS),
             item_embedding.T.reshape(8, 8, NROWS),
             user_bias.reshape(user_bias.shape[0]),
             item_bias.reshape(item_bias.shape[0]),
             global_bias)


# speedup vs baseline: 22.7540x; 1.1675x over previous
import functools

import jax
import jax.numpy as jnp
from jax import lax
from jax.experimental import pallas as pl
from jax.experimental.pallas import tpu as pltpu
from jax.experimental.pallas import tpu_sc as plsc

BATCH = 16384
NROWS = 1000000
EMBED_DIM = 64
LANES = 16
IDX_CHUNK = 128
NSLAB = 7813
SLABS_PER_TILE = 244
LIST_CAP = 4112
HIT_CAP = 528
OUT_CAP = 1024


def _sweep_body(users_ref, items_ref, ue3_ref, ie3_ref, outu_ref, outi_ref,
                allu_v, alli_v, lval, lpos, hitval, hitpos, slabA, slabB,
                outbuf, semA, semB, sem_out, *, num_cores):
    wid = lax.axis_index("s") * num_cores + lax.axis_index("c")
    lo = wid * SLABS_PER_TILE
    extra_s = jnp.where(wid < 5, NSLAB - 5 + wid, 0)

    pltpu.sync_copy(users_ref, allu_v)
    pltpu.sync_copy(items_ref, alli_v)

    lane = lax.iota(jnp.int32, LANES)
    chiv = [(m * LANES + lane) >> 3 for m in range(EMBED_DIM // LANES)]
    clov = [(m * LANES + lane) & 7 for m in range(EMBED_DIM // LANES)]

    def sweep_table(idx_v, tab_ref, out1_ref):
        def clr(ch, _):
            lval[pl.ds(ch * LANES, LANES)] = lax.broadcast(
                jnp.int32(-1), (LANES,))
            return 0
        lax.fori_loop(0, LIST_CAP // LANES, clr, 0)

        def build(ch, cnt):
            sl = pl.ds(ch * LANES, LANES)
            vals = idx_v[sl]
            slabv = vals >> 7
            m = ((slabv >= lo) & (slabv < lo + SLABS_PER_TILE)
                 | (slabv == extra_s))
            plsc.store_compressed(lval.at[pl.ds(cnt, LANES)], vals, mask=m)
            plsc.store_compressed(lpos.at[pl.ds(cnt, LANES)],
                                  ch * LANES + lane, mask=m)
            return cnt + plsc.all_reduce_population_count(m)[0]
        cnt = lax.fori_loop(0, BATCH // LANES, build, jnp.int32(0))
        nscan = (cnt + LANES - 1) >> 4

        def fire(buf, s, sem):
            start = pl.multiple_of(s * IDX_CHUNK, IDX_CHUNK)
            return pltpu.async_copy(
                tab_ref.at[:, :, pl.ds(start, IDX_CHUNK)], buf, sem)

        def wait(buf, sem):
            pltpu.make_async_copy(
                tab_ref.at[:, :, pl.ds(0, IDX_CHUNK)], buf, sem).wait()

        def process(s, buf, wcnt):
            def scan(ch2, hcnt):
                sl = pl.ds(ch2 * LANES, LANES)
                vals = lval[sl]
                pv = lpos[sl]
                m = (vals >> 7) == s
                plsc.store_compressed(hitval.at[pl.ds(hcnt, LANES)], vals, mask=m)
                plsc.store_compressed(hitpos.at[pl.ds(hcnt, LANES)], pv, mask=m)
                return hcnt + plsc.all_reduce_population_count(m)[0]
            hcnt = lax.fori_loop(0, nscan, scan, jnp.int32(0))

            def extract(h, w):
                hb = (h >> 4) << 4
                lsel = lane == (h & (LANES - 1))
                hv = hitval[pl.ds(hb, LANES)]
                val = jnp.sum(jnp.where(lsel, hv, 0))
                pv = hitpos[pl.ds(hb, LANES)]
                pos = jnp.sum(jnp.where(lsel, pv, 0))
                rm = lax.broadcast(val & (IDX_CHUNK - 1), (LANES,))
                for mch in range(EMBED_DIM // LANES):
                    g16 = plsc.load_gather(buf, [chiv[mch], clov[mch], rm])
                    outbuf[pl.ds(w * EMBED_DIM + mch * LANES, LANES)] = g16
                pltpu.async_copy(
                    outbuf.at[pl.ds(w * EMBED_DIM, EMBED_DIM)],
                    out1_ref.at[pl.ds(pos * EMBED_DIM, EMBED_DIM)], sem_out)
                return w + 1
            return lax.fori_loop(0, hcnt, extract, wcnt)

        fire(slabA, lo, semA)

        def pair(ss, wcnt):
            s0 = lo + 2 * ss
            fire(slabB, s0 + 1, semB)
            wait(slabA, semA)
            wcnt = process(s0, slabA, wcnt)

            @pl.when(ss < SLABS_PER_TILE // 2 - 1)
            def _():
                fire(slabA, s0 + 2, semA)
            wait(slabB, semB)
            wcnt = process(s0 + 1, slabB, wcnt)
            return wcnt

        wcnt = lax.fori_loop(0, SLABS_PER_TILE // 2, pair, jnp.int32(0))

        fire(slabA, extra_s, semA)
        wait(slabA, semA)
        process(extra_s, slabA, wcnt)

        def drain(d, _):
            pltpu.make_async_copy(
                outbuf.at[pl.ds(0, EMBED_DIM)],
                out1_ref.at[pl.ds(0, EMBED_DIM)], sem_out).wait()
            return 0
        lax.fori_loop(0, cnt, drain, 0)

    sweep_table(allu_v, ue3_ref, outu_ref)
    sweep_table(alli_v, ie3_ref, outi_ref)


def _dot_body(users_ref, items_ref, outu_ref, outi_ref, ub_ref, ib_ref,
              gb_ref, out_ref, uidx_v, iidx_v, urows1, irows1,
              ub_v, ib_v, scores_v, gb_v, sem_bias, *, b_per_w, num_cores):
    wid = lax.axis_index("s") * num_cores + lax.axis_index("c")
    base = wid * b_per_w
    nchunk = b_per_w // IDX_CHUNK

    pltpu.sync_copy(users_ref.at[pl.ds(base, b_per_w)], uidx_v)
    pltpu.sync_copy(items_ref.at[pl.ds(base, b_per_w)], iidx_v)
    pltpu.sync_copy(gb_ref, gb_v.at[pl.ds(0, 1)])
    pltpu.sync_copy(outu_ref.at[pl.ds(base * EMBED_DIM, b_per_w * EMBED_DIM)],
                    urows1)
    pltpu.sync_copy(outi_ref.at[pl.ds(base * EMBED_DIM, b_per_w * EMBED_DIM)],
                    irows1)

    bias_handles = []
    for j in range(nchunk):
        sl = pl.ds(j * IDX_CHUNK, IDX_CHUNK)
        bias_handles.append(pltpu.async_copy(
            ub_ref.at[uidx_v.at[sl]], ub_v.at[sl], sem_bias))
        bias_handles.append(pltpu.async_copy(
            ib_ref.at[iidx_v.at[sl]], ib_v.at[sl], sem_bias))
    for h in bias_handles:
        h.wait()

    gb = gb_v[pl.ds(0, LANES)][0]
    lane = lax.iota(jnp.int32, LANES)

    def group(g, _):
        sl = pl.ds(g * LANES, LANES)
        score_vec = jnp.zeros((LANES,), jnp.float32)
        for j in range(LANES):
            e0 = (g * LANES + j) * EMBED_DIM
            acc = None
            for m in range(EMBED_DIM // LANES):
                u16 = urows1[pl.ds(e0 + m * LANES, LANES)]
                i16 = irows1[pl.ds(e0 + m * LANES, LANES)]
                p = u16 * i16
                acc = p if acc is None else acc + p
            s_val = jnp.sum(acc)
            score_vec = jnp.where(lane == j, s_val, score_vec)
        scores_v[sl] = score_vec + ub_v[sl] + ib_v[sl] + gb
        return 0

    lax.fori_loop(0, b_per_w // LANES, group, 0)

    pltpu.sync_copy(scores_v, out_ref.at[pl.ds(base, b_per_w)])


def kernel(users, items, user_embedding, item_embedding, user_bias,
           item_bias, global_bias):
    info = plsc.get_sparse_core_info()
    num_workers = info.num_cores * info.num_subcores
    b_per_w = BATCH // num_workers
    cparams = pltpu.CompilerParams(use_tc_tiling_on_sc=True,
                                   needs_layout_passes=False)
    mesh = plsc.VectorSubcoreMesh(core_axis_name="c", subcore_axis_name="s")

    sweep = pl.kernel(
        functools.partial(_sweep_body, num_cores=info.num_cores),
        mesh=mesh,
        compiler_params=cparams,
        out_type=(jax.ShapeDtypeStruct((BATCH * EMBED_DIM,), jnp.float32),
                  jax.ShapeDtypeStruct((BATCH * EMBED_DIM,), jnp.float32)),
        scratch_types=[
            pltpu.VMEM((BATCH,), jnp.int32),
            pltpu.VMEM((BATCH,), jnp.int32),
            pltpu.VMEM((LIST_CAP,), jnp.int32),
            pltpu.VMEM((LIST_CAP,), jnp.int32),
            pltpu.VMEM((HIT_CAP,), jnp.int32),
            pltpu.VMEM((HIT_CAP,), jnp.int32),
            pltpu.VMEM((8, 8, IDX_CHUNK), jnp.float32),
            pltpu.VMEM((8, 8, IDX_CHUNK), jnp.float32),
            pltpu.VMEM((OUT_CAP * EMBED_DIM,), jnp.float32),
            pltpu.SemaphoreType.DMA,
            pltpu.SemaphoreType.DMA,
            pltpu.SemaphoreType.DMA,
        ],
    )

    dot = pl.kernel(
        functools.partial(_dot_body, b_per_w=b_per_w,
                          num_cores=info.num_cores),
        mesh=mesh,
        compiler_params=cparams,
        out_type=jax.ShapeDtypeStruct((BATCH,), jnp.float32),
        scratch_types=[
            pltpu.VMEM((b_per_w,), jnp.int32),
            pltpu.VMEM((b_per_w,), jnp.int32),
            pltpu.VMEM((b_per_w * EMBED_DIM,), jnp.float32),
            pltpu.VMEM((b_per_w * EMBED_DIM,), jnp.float32),
            pltpu.VMEM((b_per_w,), jnp.float32),
            pltpu.VMEM((b_per_w,), jnp.float32),
            pltpu.VMEM((b_per_w,), jnp.float32),
            pltpu.VMEM((LANES,), jnp.float32),
            pltpu.SemaphoreType.DMA,
        ],
    )

    u32 = users.astype(jnp.int32)
    i32 = items.astype(jnp.int32)
    rows_u, rows_i = sweep(u32, i32,
                           user_embedding.T.reshape(8, 8, NROWS),
                           item_embedding.T.reshape(8, 8, NROWS))
    return dot(u32, i32, rows_u, rows_i,
               user_bias.reshape(user_bias.shape[0]),
               item_bias.reshape(item_bias.shape[0]),
               global_bias)


# speedup vs baseline: 30.1687x; 1.3259x over previous
import functools

import jax
import jax.numpy as jnp
from jax import lax
from jax.experimental import pallas as pl
from jax.experimental.pallas import tpu as pltpu
from jax.experimental.pallas import tpu_sc as plsc

BATCH = 16384
NROWS = 1000000
EMBED_DIM = 64
LANES = 16
IDX_CHUNK = 128
NSLAB = 7813
SLABS_PER_TILE = 244
LIST_CAP = 4112
HIT_CAP = 528
OUT_CAP = 768


def _sweep_body(users_ref, items_ref, ue3_ref, ie3_ref, outu_ref, outi_ref,
                allu_v, alli_v, lval, lpos, hitval, hitpos, slabA, slabB,
                slabC, slabD, outbuf, semA, semB, semC, semD, sem_out, *,
                num_cores):
    wid = lax.axis_index("s") * num_cores + lax.axis_index("c")
    lo = wid * SLABS_PER_TILE
    extra_s = jnp.where(wid < 5, NSLAB - 5 + wid, 0)

    pltpu.sync_copy(users_ref, allu_v)
    pltpu.sync_copy(items_ref, alli_v)

    lane = lax.iota(jnp.int32, LANES)
    chiv = [(m * LANES + lane) >> 3 for m in range(EMBED_DIM // LANES)]
    clov = [(m * LANES + lane) & 7 for m in range(EMBED_DIM // LANES)]

    def sweep_table(idx_v, tab_ref, out1_ref):
        def clr(ch, _):
            lval[pl.ds(ch * LANES, LANES)] = lax.broadcast(
                jnp.int32(-1), (LANES,))
            return 0
        lax.fori_loop(0, LIST_CAP // LANES, clr, 0)

        def build(ch, cnt):
            sl = pl.ds(ch * LANES, LANES)
            vals = idx_v[sl]
            slabv = vals >> 7
            m = ((slabv >= lo) & (slabv < lo + SLABS_PER_TILE)
                 | (slabv == extra_s))
            plsc.store_compressed(lval.at[pl.ds(cnt, LANES)], vals, mask=m)
            plsc.store_compressed(lpos.at[pl.ds(cnt, LANES)],
                                  ch * LANES + lane, mask=m)
            return cnt + plsc.all_reduce_population_count(m)[0]
        cnt = lax.fori_loop(0, BATCH // LANES, build, jnp.int32(0))
        nscan = (cnt + LANES - 1) >> 4

        def fire(buf, s, sem):
            start = pl.multiple_of(s * IDX_CHUNK, IDX_CHUNK)
            return pltpu.async_copy(
                tab_ref.at[:, :, pl.ds(start, IDX_CHUNK)], buf, sem)

        def wait(buf, sem):
            pltpu.make_async_copy(
                tab_ref.at[:, :, pl.ds(0, IDX_CHUNK)], buf, sem).wait()

        def process(s, buf, wcnt):
            def scan(ch2, hcnt):
                sl = pl.ds(ch2 * LANES, LANES)
                vals = lval[sl]
                pv = lpos[sl]
                m = (vals >> 7) == s
                plsc.store_compressed(hitval.at[pl.ds(hcnt, LANES)], vals, mask=m)
                plsc.store_compressed(hitpos.at[pl.ds(hcnt, LANES)], pv, mask=m)
                return hcnt + plsc.all_reduce_population_count(m)[0]
            hcnt = lax.fori_loop(0, nscan, scan, jnp.int32(0))

            def extract(h, w):
                hb = (h >> 4) << 4
                lsel = lane == (h & (LANES - 1))
                hv = hitval[pl.ds(hb, LANES)]
                val = jnp.sum(jnp.where(lsel, hv, 0))
                pv = hitpos[pl.ds(hb, LANES)]
                pos = jnp.sum(jnp.where(lsel, pv, 0))
                rm = lax.broadcast(val & (IDX_CHUNK - 1), (LANES,))
                for mch in range(EMBED_DIM // LANES):
                    g16 = plsc.load_gather(buf, [chiv[mch], clov[mch], rm])
                    outbuf[pl.ds(w * EMBED_DIM + mch * LANES, LANES)] = g16
                pltpu.async_copy(
                    outbuf.at[pl.ds(w * EMBED_DIM, EMBED_DIM)],
                    out1_ref.at[pl.ds(pos * EMBED_DIM, EMBED_DIM)], sem_out)
                return w + 1
            return lax.fori_loop(0, hcnt, extract, wcnt)

        slabs = [slabA, slabB, slabC, slabD]
        sems = [semA, semB, semC, semD]
        nquad = SLABS_PER_TILE // 4
        for k in range(3):
            fire(slabs[k], lo + k, sems[k])

        def quad(qq, wcnt):
            s = lo + 4 * qq
            for k in range(4):
                if k == 0:
                    fire(slabs[3], s + 3, sems[3])
                else:
                    @pl.when(qq < nquad - 1)
                    def _(k=k):
                        fire(slabs[k - 1], s + 3 + k, sems[k - 1])
                wait(slabs[k], sems[k])
                wcnt = process(s + k, slabs[k], wcnt)
            return wcnt

        wcnt = lax.fori_loop(0, nquad, quad, jnp.int32(0))

        fire(slabA, extra_s, semA)
        wait(slabA, semA)
        process(extra_s, slabA, wcnt)

        def drain(d, _):
            pltpu.make_async_copy(
                outbuf.at[pl.ds(0, EMBED_DIM)],
                out1_ref.at[pl.ds(0, EMBED_DIM)], sem_out).wait()
            return 0
        lax.fori_loop(0, cnt, drain, 0)

    sweep_table(allu_v, ue3_ref, outu_ref)
    sweep_table(alli_v, ie3_ref, outi_ref)


def _dot_body(users_ref, items_ref, outu_ref, outi_ref, ub_ref, ib_ref,
              gb_ref, out_ref, uidx_v, iidx_v, urows1, irows1,
              ub_v, ib_v, scores_v, gb_v, sem_bias, *, b_per_w, num_cores):
    wid = lax.axis_index("s") * num_cores + lax.axis_index("c")
    base = wid * b_per_w
    nchunk = b_per_w // IDX_CHUNK

    pltpu.sync_copy(users_ref.at[pl.ds(base, b_per_w)], uidx_v)
    pltpu.sync_copy(items_ref.at[pl.ds(base, b_per_w)], iidx_v)
    pltpu.sync_copy(gb_ref, gb_v.at[pl.ds(0, 1)])
    pltpu.sync_copy(outu_ref.at[pl.ds(base * EMBED_DIM, b_per_w * EMBED_DIM)],
                    urows1)
    pltpu.sync_copy(outi_ref.at[pl.ds(base * EMBED_DIM, b_per_w * EMBED_DIM)],
                    irows1)

    bias_handles = []
    for j in range(nchunk):
        sl = pl.ds(j * IDX_CHUNK, IDX_CHUNK)
        bias_handles.append(pltpu.async_copy(
            ub_ref.at[uidx_v.at[sl]], ub_v.at[sl], sem_bias))
        bias_handles.append(pltpu.async_copy(
            ib_ref.at[iidx_v.at[sl]], ib_v.at[sl], sem_bias))
    for h in bias_handles:
        h.wait()

    gb = gb_v[pl.ds(0, LANES)][0]
    lane = lax.iota(jnp.int32, LANES)

    def group(g, _):
        sl = pl.ds(g * LANES, LANES)
        score_vec = jnp.zeros((LANES,), jnp.float32)
        for j in range(LANES):
            e0 = (g * LANES + j) * EMBED_DIM
            acc = None
            for m in range(EMBED_DIM // LANES):
                u16 = urows1[pl.ds(e0 + m * LANES, LANES)]
                i16 = irows1[pl.ds(e0 + m * LANES, LANES)]
                p = u16 * i16
                acc = p if acc is None else acc + p
            s_val = jnp.sum(acc)
            score_vec = jnp.where(lane == j, s_val, score_vec)
        scores_v[sl] = score_vec + ub_v[sl] + ib_v[sl] + gb
        return 0

    lax.fori_loop(0, b_per_w // LANES, group, 0)

    pltpu.sync_copy(scores_v, out_ref.at[pl.ds(base, b_per_w)])


def kernel(users, items, user_embedding, item_embedding, user_bias,
           item_bias, global_bias):
    info = plsc.get_sparse_core_info()
    num_workers = info.num_cores * info.num_subcores
    b_per_w = BATCH // num_workers
    cparams = pltpu.CompilerParams(use_tc_tiling_on_sc=True,
                                   needs_layout_passes=False)
    mesh = plsc.VectorSubcoreMesh(core_axis_name="c", subcore_axis_name="s")

    sweep = pl.kernel(
        functools.partial(_sweep_body, num_cores=info.num_cores),
        mesh=mesh,
        compiler_params=cparams,
        out_type=(jax.ShapeDtypeStruct((BATCH * EMBED_DIM,), jnp.float32),
                  jax.ShapeDtypeStruct((BATCH * EMBED_DIM,), jnp.float32)),
        scratch_types=[
            pltpu.VMEM((BATCH,), jnp.int32),
            pltpu.VMEM((BATCH,), jnp.int32),
            pltpu.VMEM((LIST_CAP,), jnp.int32),
            pltpu.VMEM((LIST_CAP,), jnp.int32),
            pltpu.VMEM((HIT_CAP,), jnp.int32),
            pltpu.VMEM((HIT_CAP,), jnp.int32),
            pltpu.VMEM((8, 8, IDX_CHUNK), jnp.float32),
            pltpu.VMEM((8, 8, IDX_CHUNK), jnp.float32),
            pltpu.VMEM((8, 8, IDX_CHUNK), jnp.float32),
            pltpu.VMEM((8, 8, IDX_CHUNK), jnp.float32),
            pltpu.VMEM((OUT_CAP * EMBED_DIM,), jnp.float32),
            pltpu.SemaphoreType.DMA,
            pltpu.SemaphoreType.DMA,
            pltpu.SemaphoreType.DMA,
            pltpu.SemaphoreType.DMA,
            pltpu.SemaphoreType.DMA,
        ],
    )

    dot = pl.kernel(
        functools.partial(_dot_body, b_per_w=b_per_w,
                          num_cores=info.num_cores),
        mesh=mesh,
        compiler_params=cparams,
        out_type=jax.ShapeDtypeStruct((BATCH,), jnp.float32),
        scratch_types=[
            pltpu.VMEM((b_per_w,), jnp.int32),
            pltpu.VMEM((b_per_w,), jnp.int32),
            pltpu.VMEM((b_per_w * EMBED_DIM,), jnp.float32),
            pltpu.VMEM((b_per_w * EMBED_DIM,), jnp.float32),
            pltpu.VMEM((b_per_w,), jnp.float32),
            pltpu.VMEM((b_per_w,), jnp.float32),
            pltpu.VMEM((b_per_w,), jnp.float32),
            pltpu.VMEM((LANES,), jnp.float32),
            pltpu.SemaphoreType.DMA,
        ],
    )

    u32 = users.astype(jnp.int32)
    i32 = items.astype(jnp.int32)
    rows_u, rows_i = sweep(u32, i32,
                           user_embedding.T.reshape(8, 8, NROWS),
                           item_embedding.T.reshape(8, 8, NROWS))
    return dot(u32, i32, rows_u, rows_i,
               user_bias.reshape(user_bias.shape[0]),
               item_bias.reshape(item_bias.shape[0]),
               global_bias)


# speedup vs baseline: 34.8944x; 1.1566x over previous
import functools

import jax
import jax.numpy as jnp
from jax import lax
from jax.experimental import pallas as pl
from jax.experimental.pallas import tpu as pltpu
from jax.experimental.pallas import tpu_sc as plsc

BATCH = 16384
NROWS = 1000000
EMBED_DIM = 64
LANES = 16
IDX_CHUNK = 128
NSLAB = 7813
SLABS_PER_TILE = 244
LIST_CAP = 4112
NGROUP = 16
GROUP_CAP = 256
HIT_CAP = 528
OUT_CAP = 768


def _sweep_body(users_ref, items_ref, ue3_ref, ie3_ref, outu_ref, outi_ref,
                all_v, lval, lpos, gval, gpos, hitval, hitpos, slabA, slabB,
                slabC, slabD, outbuf, semA, semB, semC, semD, sem_out, *,
                num_cores):
    wid = lax.axis_index("s") * num_cores + lax.axis_index("c")
    lo = wid * SLABS_PER_TILE
    extra_s = jnp.where(wid < 5, NSLAB - 5 + wid, 0)

    lane = lax.iota(jnp.int32, LANES)
    chiv = [(m * LANES + lane) >> 3 for m in range(EMBED_DIM // LANES)]
    clov = [(m * LANES + lane) & 7 for m in range(EMBED_DIM // LANES)]

    def sweep_table(idx_ref, tab_ref, out1_ref):
        pltpu.sync_copy(idx_ref, all_v)

        def clr(ch, _):
            lval[pl.ds(ch * LANES, LANES)] = lax.broadcast(
                jnp.int32(-1), (LANES,))
            return 0
        lax.fori_loop(0, LIST_CAP // LANES, clr, 0)

        def clrg(ch, _):
            gval[pl.ds(ch * LANES, LANES)] = lax.broadcast(
                jnp.int32(-1), (LANES,))
            return 0
        lax.fori_loop(0, (NGROUP + 1) * GROUP_CAP // LANES, clrg, 0)

        def build(ch, cnt):
            sl = pl.ds(ch * LANES, LANES)
            vals = all_v[sl]
            slabv = vals >> 7
            m = ((slabv >= lo) & (slabv < lo + SLABS_PER_TILE)
                 | (slabv == extra_s))
            plsc.store_compressed(lval.at[pl.ds(cnt, LANES)], vals, mask=m)
            plsc.store_compressed(lpos.at[pl.ds(cnt, LANES)],
                                  ch * LANES + lane, mask=m)
            return cnt + plsc.all_reduce_population_count(m)[0]
        cnt = lax.fori_loop(0, BATCH // LANES, build, jnp.int32(0))
        nscan = (cnt + LANES - 1) >> 4

        gcnt_vec = jnp.zeros((LANES,), jnp.int32)
        extra_cnt = jnp.int32(0)
        for g in range(NGROUP + 1):
            def bing(ch, c2, g=g):
                sl = pl.ds(ch * LANES, LANES)
                vals = lval[sl]
                pv = lpos[sl]
                if g == NGROUP:
                    m = (vals >> 7) == extra_s
                else:
                    m = ((vals >> 7) - lo) >> 4 == g
                plsc.store_compressed(
                    gval.at[pl.ds(g * GROUP_CAP + c2, LANES)], vals, mask=m)
                plsc.store_compressed(
                    gpos.at[pl.ds(g * GROUP_CAP + c2, LANES)], pv, mask=m)
                return c2 + plsc.all_reduce_population_count(m)[0]
            cg = lax.fori_loop(0, nscan, bing, jnp.int32(0))
            if g == NGROUP:
                extra_cnt = cg
            else:
                gcnt_vec = jnp.where(lane == g, cg, gcnt_vec)

        def fire(buf, s, sem):
            start = pl.multiple_of(s * IDX_CHUNK, IDX_CHUNK)
            return pltpu.async_copy(
                tab_ref.at[:, :, pl.ds(start, IDX_CHUNK)], buf, sem)

        def wait(buf, sem):
            pltpu.make_async_copy(
                tab_ref.at[:, :, pl.ds(0, IDX_CHUNK)], buf, sem).wait()

        def process(s, buf, wcnt, gbase, gn):
            def scan(ch2, hcnt):
                sl = pl.ds(gbase + ch2 * LANES, LANES)
                vals = gval[sl]
                pv = gpos[sl]
                m = (vals >> 7) == s
                plsc.store_compressed(hitval.at[pl.ds(hcnt, LANES)], vals, mask=m)
                plsc.store_compressed(hitpos.at[pl.ds(hcnt, LANES)], pv, mask=m)
                return hcnt + plsc.all_reduce_population_count(m)[0]
            hcnt = lax.fori_loop(0, gn, scan, jnp.int32(0))

            def extract(h, w):
                hb = (h >> 4) << 4
                lsel = lane == (h & (LANES - 1))
                hv = hitval[pl.ds(hb, LANES)]
                val = jnp.sum(jnp.where(lsel, hv, 0))
                pv = hitpos[pl.ds(hb, LANES)]
                pos = jnp.sum(jnp.where(lsel, pv, 0))
                rm = lax.broadcast(val & (IDX_CHUNK - 1), (LANES,))
                for mch in range(EMBED_DIM // LANES):
                    g16 = plsc.load_gather(buf, [chiv[mch], clov[mch], rm])
                    outbuf[pl.ds(w * EMBED_DIM + mch * LANES, LANES)] = g16
                pltpu.async_copy(
                    outbuf.at[pl.ds(w * EMBED_DIM, EMBED_DIM)],
                    out1_ref.at[pl.ds(pos * EMBED_DIM, EMBED_DIM)], sem_out)
                return w + 1
            return lax.fori_loop(0, hcnt, extract, wcnt)

        slabs = [slabA, slabB, slabC, slabD]
        sems = [semA, semB, semC, semD]
        nquad = SLABS_PER_TILE // 4
        for k in range(3):
            fire(slabs[k], lo + k, sems[k])

        def quad(qq, wcnt):
            s = lo + 4 * qq
            for k in range(4):
                if k == 0:
                    fire(slabs[3], s + 3, sems[3])
                else:
                    @pl.when(qq < nquad - 1)
                    def _(k=k):
                        fire(slabs[k - 1], s + 3 + k, sems[k - 1])
                wait(slabs[k], sems[k])
                gid = (s + k - lo) >> 4
                gc = jnp.sum(jnp.where(lane == gid, gcnt_vec, 0))
                wcnt = process(s + k, slabs[k], wcnt, gid * GROUP_CAP,
                               (gc + LANES - 1) >> 4)
            return wcnt

        wcnt = lax.fori_loop(0, nquad, quad, jnp.int32(0))

        fire(slabA, extra_s, semA)
        wait(slabA, semA)
        process(extra_s, slabA, wcnt, NGROUP * GROUP_CAP,
                (extra_cnt + LANES - 1) >> 4)

        def drain(d, _):
            pltpu.make_async_copy(
                outbuf.at[pl.ds(0, EMBED_DIM)],
                out1_ref.at[pl.ds(0, EMBED_DIM)], sem_out).wait()
            return 0
        lax.fori_loop(0, cnt, drain, 0)

    sweep_table(users_ref, ue3_ref, outu_ref)
    sweep_table(items_ref, ie3_ref, outi_ref)


def _dot_body(users_ref, items_ref, outu_ref, outi_ref, ub_ref, ib_ref,
              gb_ref, out_ref, uidx_v, iidx_v, urows1, irows1,
              ub_v, ib_v, scores_v, gb_v, sem_bias, *, b_per_w, num_cores):
    wid = lax.axis_index("s") * num_cores + lax.axis_index("c")
    base = wid * b_per_w
    nchunk = b_per_w // IDX_CHUNK

    pltpu.sync_copy(users_ref.at[pl.ds(base, b_per_w)], uidx_v)
    pltpu.sync_copy(items_ref.at[pl.ds(base, b_per_w)], iidx_v)
    pltpu.sync_copy(gb_ref, gb_v.at[pl.ds(0, 1)])
    pltpu.sync_copy(outu_ref.at[pl.ds(base * EMBED_DIM, b_per_w * EMBED_DIM)],
                    urows1)
    pltpu.sync_copy(outi_ref.at[pl.ds(base * EMBED_DIM, b_per_w * EMBED_DIM)],
                    irows1)

    bias_handles = []
    for j in range(nchunk):
        sl = pl.ds(j * IDX_CHUNK, IDX_CHUNK)
        bias_handles.append(pltpu.async_copy(
            ub_ref.at[uidx_v.at[sl]], ub_v.at[sl], sem_bias))
        bias_handles.append(pltpu.async_copy(
            ib_ref.at[iidx_v.at[sl]], ib_v.at[sl], sem_bias))
    for h in bias_handles:
        h.wait()

    gb = gb_v[pl.ds(0, LANES)][0]
    lane = lax.iota(jnp.int32, LANES)

    def group(g, _):
        sl = pl.ds(g * LANES, LANES)
        score_vec = jnp.zeros((LANES,), jnp.float32)
        for j in range(LANES):
            e0 = (g * LANES + j) * EMBED_DIM
            acc = None
            for m in range(EMBED_DIM // LANES):
                u16 = urows1[pl.ds(e0 + m * LANES, LANES)]
                i16 = irows1[pl.ds(e0 + m * LANES, LANES)]
                p = u16 * i16
                acc = p if acc is None else acc + p
            s_val = jnp.sum(acc)
            score_vec = jnp.where(lane == j, s_val, score_vec)
        scores_v[sl] = score_vec + ub_v[sl] + ib_v[sl] + gb
        return 0

    lax.fori_loop(0, b_per_w // LANES, group, 0)

    pltpu.sync_copy(scores_v, out_ref.at[pl.ds(base, b_per_w)])


def kernel(users, items, user_embedding, item_embedding, user_bias,
           item_bias, global_bias):
    info = plsc.get_sparse_core_info()
    num_workers = info.num_cores * info.num_subcores
    b_per_w = BATCH // num_workers
    cparams = pltpu.CompilerParams(use_tc_tiling_on_sc=True,
                                   needs_layout_passes=False)
    mesh = plsc.VectorSubcoreMesh(core_axis_name="c", subcore_axis_name="s")

    sweep = pl.kernel(
        functools.partial(_sweep_body, num_cores=info.num_cores),
        mesh=mesh,
        compiler_params=cparams,
        out_type=(jax.ShapeDtypeStruct((BATCH * EMBED_DIM,), jnp.float32),
                  jax.ShapeDtypeStruct((BATCH * EMBED_DIM,), jnp.float32)),
        scratch_types=[
            pltpu.VMEM((BATCH,), jnp.int32),
            pltpu.VMEM((LIST_CAP,), jnp.int32),
            pltpu.VMEM((LIST_CAP,), jnp.int32),
            pltpu.VMEM(((NGROUP + 1) * GROUP_CAP,), jnp.int32),
            pltpu.VMEM(((NGROUP + 1) * GROUP_CAP,), jnp.int32),
            pltpu.VMEM((HIT_CAP,), jnp.int32),
            pltpu.VMEM((HIT_CAP,), jnp.int32),
            pltpu.VMEM((8, 8, IDX_CHUNK), jnp.float32),
            pltpu.VMEM((8, 8, IDX_CHUNK), jnp.float32),
            pltpu.VMEM((8, 8, IDX_CHUNK), jnp.float32),
            pltpu.VMEM((8, 8, IDX_CHUNK), jnp.float32),
            pltpu.VMEM((OUT_CAP * EMBED_DIM,), jnp.float32),
            pltpu.SemaphoreType.DMA,
            pltpu.SemaphoreType.DMA,
            pltpu.SemaphoreType.DMA,
            pltpu.SemaphoreType.DMA,
            pltpu.SemaphoreType.DMA,
        ],
    )

    dot = pl.kernel(
        functools.partial(_dot_body, b_per_w=b_per_w,
                          num_cores=info.num_cores),
        mesh=mesh,
        compiler_params=cparams,
        out_type=jax.ShapeDtypeStruct((BATCH,), jnp.float32),
        scratch_types=[
            pltpu.VMEM((b_per_w,), jnp.int32),
            pltpu.VMEM((b_per_w,), jnp.int32),
            pltpu.VMEM((b_per_w * EMBED_DIM,), jnp.float32),
            pltpu.VMEM((b_per_w * EMBED_DIM,), jnp.float32),
            pltpu.VMEM((b_per_w,), jnp.float32),
            pltpu.VMEM((b_per_w,), jnp.float32),
            pltpu.VMEM((b_per_w,), jnp.float32),
            pltpu.VMEM((LANES,), jnp.float32),
            pltpu.SemaphoreType.DMA,
        ],
    )

    u32 = users.astype(jnp.int32)
    i32 = items.astype(jnp.int32)
    rows_u, rows_i = sweep(u32, i32,
                           user_embedding.T.reshape(8, 8, NROWS),
                           item_embedding.T.reshape(8, 8, NROWS))
    return dot(u32, i32, rows_u, rows_i,
               user_bias.reshape(user_bias.shape[0]),
               item_bias.reshape(item_bias.shape[0]),
               global_bias)


# speedup vs baseline: 35.8584x; 1.0276x over previous
import functools

import jax
import jax.numpy as jnp
from jax import lax
from jax.experimental import pallas as pl
from jax.experimental.pallas import tpu as pltpu
from jax.experimental.pallas import tpu_sc as plsc

BATCH = 16384
NROWS = 1000000
EMBED_DIM = 64
LANES = 16
IDX_CHUNK = 128
NSLAB = 7813
SLABS_PER_TILE = 244
LIST_CAP = 4112
NGROUP = 16
GROUP_CAP = 256
HIT_CAP = 528
OUT_CAP = 768


def _sweep_body(users_ref, items_ref, ue3_ref, ie3_ref, outu_ref, outi_ref,
                all_v, lval, lpos, gval, gpos, hitval, hitpos, slabA, slabB,
                slabC, slabD, outbuf, semA, semB, semC, semD, sem_out, *,
                num_cores):
    wid = lax.axis_index("s") * num_cores + lax.axis_index("c")
    lo = wid * SLABS_PER_TILE
    extra_s = jnp.where(wid < 5, NSLAB - 5 + wid, 0)

    lane = lax.iota(jnp.int32, LANES)
    chiv = [(m * LANES + lane) >> 3 for m in range(EMBED_DIM // LANES)]
    clov = [(m * LANES + lane) & 7 for m in range(EMBED_DIM // LANES)]

    def sweep_table(idx_ref, tab_ref, out1_ref):
        pltpu.sync_copy(idx_ref, all_v)

        def clr(ch, _):
            lval[pl.ds(ch * LANES, LANES)] = lax.broadcast(
                jnp.int32(-1), (LANES,))
            return 0
        lax.fori_loop(0, LIST_CAP // LANES, clr, 0)

        def clrg(ch, _):
            gval[pl.ds(ch * LANES, LANES)] = lax.broadcast(
                jnp.int32(-1), (LANES,))
            return 0
        lax.fori_loop(0, (NGROUP + 1) * GROUP_CAP // LANES, clrg, 0)

        def build(ch, cnt):
            sl = pl.ds(ch * LANES, LANES)
            vals = all_v[sl]
            slabv = vals >> 7
            m = ((slabv >= lo) & (slabv < lo + SLABS_PER_TILE)
                 | (slabv == extra_s))
            plsc.store_compressed(lval.at[pl.ds(cnt, LANES)], vals, mask=m)
            plsc.store_compressed(lpos.at[pl.ds(cnt, LANES)],
                                  ch * LANES + lane, mask=m)
            return cnt + plsc.all_reduce_population_count(m)[0]
        cnt = lax.fori_loop(0, BATCH // LANES, build, jnp.int32(0))
        nscan = (cnt + LANES - 1) >> 4

        gcnt_vec = jnp.zeros((LANES,), jnp.int32)
        extra_cnt = jnp.int32(0)
        for g in range(NGROUP + 1):
            def bing(ch, c2, g=g):
                sl = pl.ds(ch * LANES, LANES)
                vals = lval[sl]
                pv = lpos[sl]
                if g == NGROUP:
                    m = (vals >> 7) == extra_s
                else:
                    m = ((vals >> 7) - lo) >> 4 == g
                plsc.store_compressed(
                    gval.at[pl.ds(g * GROUP_CAP + c2, LANES)], vals, mask=m)
                plsc.store_compressed(
                    gpos.at[pl.ds(g * GROUP_CAP + c2, LANES)], pv, mask=m)
                return c2 + plsc.all_reduce_population_count(m)[0]
            cg = lax.fori_loop(0, nscan, bing, jnp.int32(0))
            if g == NGROUP:
                extra_cnt = cg
            else:
                gcnt_vec = jnp.where(lane == g, cg, gcnt_vec)

        def fire(buf, s, sem):
            start = pl.multiple_of(s * IDX_CHUNK, IDX_CHUNK)
            return pltpu.async_copy(
                tab_ref.at[:, :, pl.ds(start, IDX_CHUNK)], buf, sem)

        def wait(buf, sem):
            pltpu.make_async_copy(
                tab_ref.at[:, :, pl.ds(0, IDX_CHUNK)], buf, sem).wait()

        def scan_slab(s, slot, gbase, gn):
            hb0 = slot * HIT_CAP

            def scan(ch2, hcnt):
                sl = pl.ds(gbase + ch2 * LANES, LANES)
                vals = gval[sl]
                pv = gpos[sl]
                m = (vals >> 7) == s
                plsc.store_compressed(
                    hitval.at[pl.ds(hb0 + hcnt, LANES)], vals, mask=m)
                plsc.store_compressed(
                    hitpos.at[pl.ds(hb0 + hcnt, LANES)], pv, mask=m)
                return hcnt + plsc.all_reduce_population_count(m)[0]
            return lax.fori_loop(0, gn, scan, jnp.int32(0))

        def scan_group_of(s, slot):
            gid = (s - lo) >> 4
            gc = jnp.sum(jnp.where(lane == gid, gcnt_vec, 0))
            return scan_slab(s, slot, gid * GROUP_CAP, (gc + LANES - 1) >> 4)

        def extract_slab(buf, slot, hcnt, wcnt):
            hb0 = slot * HIT_CAP

            def extract(h, w):
                hb = hb0 + ((h >> 4) << 4)
                lsel = lane == (h & (LANES - 1))
                hv = hitval[pl.ds(hb, LANES)]
                val = jnp.sum(jnp.where(lsel, hv, 0))
                pv = hitpos[pl.ds(hb, LANES)]
                pos = jnp.sum(jnp.where(lsel, pv, 0))
                rm = lax.broadcast(val & (IDX_CHUNK - 1), (LANES,))
                for mch in range(EMBED_DIM // LANES):
                    g16 = plsc.load_gather(buf, [chiv[mch], clov[mch], rm])
                    outbuf[pl.ds(w * EMBED_DIM + mch * LANES, LANES)] = g16
                pltpu.async_copy(
                    outbuf.at[pl.ds(w * EMBED_DIM, EMBED_DIM)],
                    out1_ref.at[pl.ds(pos * EMBED_DIM, EMBED_DIM)], sem_out)
                return w + 1
            return lax.fori_loop(0, hcnt, extract, wcnt)

        slabs = [slabA, slabB, slabC, slabD]
        sems = [semA, semB, semC, semD]
        nquad = SLABS_PER_TILE // 4

        def cond_fire(slot, s, h):
            @pl.when(h > 0)
            def _():
                fire(slabs[slot], s, sems[slot])

        hpre = []
        for k in range(3):
            h = scan_group_of(lo + k, k)
            cond_fire(k, lo + k, h)
            hpre.append(h)

        def quad(qq, carry):
            wcnt, h0, h1, h2, h3 = carry
            hs = [h0, h1, h2, h3]
            s = lo + 4 * qq
            for k in range(4):
                slot_fill = 3 if k == 0 else k - 1
                hn = scan_group_of(s + 3 + k, slot_fill)
                cond_fire(slot_fill, s + 3 + k, hn)
                hk = hs[k]
                hs[slot_fill] = hn

                @pl.when(hk > 0)
                def _(k=k):
                    wait(slabs[k], sems[k])
                wcnt = extract_slab(slabs[k], k, hk, wcnt)
            return (wcnt, hs[0], hs[1], hs[2], hs[3])

        carry = lax.fori_loop(
            0, nquad, quad,
            (jnp.int32(0), hpre[0], hpre[1], hpre[2], jnp.int32(0)))
        wcnt = carry[0]

        he = scan_slab(extra_s, 0, NGROUP * GROUP_CAP,
                       (extra_cnt + LANES - 1) >> 4)
        cond_fire(0, extra_s, he)

        @pl.when(he > 0)
        def _():
            wait(slabA, semA)
        extract_slab(slabA, 0, he, wcnt)

        def drain(d, _):
            pltpu.make_async_copy(
                outbuf.at[pl.ds(0, EMBED_DIM)],
                out1_ref.at[pl.ds(0, EMBED_DIM)], sem_out).wait()
            return 0
        lax.fori_loop(0, cnt, drain, 0)

    sweep_table(users_ref, ue3_ref, outu_ref)
    sweep_table(items_ref, ie3_ref, outi_ref)


def _dot_body(users_ref, items_ref, outu_ref, outi_ref, ub_ref, ib_ref,
              gb_ref, out_ref, uidx_v, iidx_v, urows1, irows1,
              ub_v, ib_v, scores_v, gb_v, sem_bias, *, b_per_w, num_cores):
    wid = lax.axis_index("s") * num_cores + lax.axis_index("c")
    base = wid * b_per_w
    nchunk = b_per_w // IDX_CHUNK

    pltpu.sync_copy(users_ref.at[pl.ds(base, b_per_w)], uidx_v)
    pltpu.sync_copy(items_ref.at[pl.ds(base, b_per_w)], iidx_v)
    pltpu.sync_copy(gb_ref, gb_v.at[pl.ds(0, 1)])
    pltpu.sync_copy(outu_ref.at[pl.ds(base * EMBED_DIM, b_per_w * EMBED_DIM)],
                    urows1)
    pltpu.sync_copy(outi_ref.at[pl.ds(base * EMBED_DIM, b_per_w * EMBED_DIM)],
                    irows1)

    bias_handles = []
    for j in range(nchunk):
        sl = pl.ds(j * IDX_CHUNK, IDX_CHUNK)
        bias_handles.append(pltpu.async_copy(
            ub_ref.at[uidx_v.at[sl]], ub_v.at[sl], sem_bias))
        bias_handles.append(pltpu.async_copy(
            ib_ref.at[iidx_v.at[sl]], ib_v.at[sl], sem_bias))
    for h in bias_handles:
        h.wait()

    gb = gb_v[pl.ds(0, LANES)][0]
    lane = lax.iota(jnp.int32, LANES)

    def group(g, _):
        sl = pl.ds(g * LANES, LANES)
        score_vec = jnp.zeros((LANES,), jnp.float32)
        for j in range(LANES):
            e0 = (g * LANES + j) * EMBED_DIM
            acc = None
            for m in range(EMBED_DIM // LANES):
                u16 = urows1[pl.ds(e0 + m * LANES, LANES)]
                i16 = irows1[pl.ds(e0 + m * LANES, LANES)]
                p = u16 * i16
                acc = p if acc is None else acc + p
            s_val = jnp.sum(acc)
            score_vec = jnp.where(lane == j, s_val, score_vec)
        scores_v[sl] = score_vec + ub_v[sl] + ib_v[sl] + gb
        return 0

    lax.fori_loop(0, b_per_w // LANES, group, 0)

    pltpu.sync_copy(scores_v, out_ref.at[pl.ds(base, b_per_w)])


def kernel(users, items, user_embedding, item_embedding, user_bias,
           item_bias, global_bias):
    info = plsc.get_sparse_core_info()
    num_workers = info.num_cores * info.num_subcores
    b_per_w = BATCH // num_workers
    cparams = pltpu.CompilerParams(use_tc_tiling_on_sc=True,
                                   needs_layout_passes=False)
    mesh = plsc.VectorSubcoreMesh(core_axis_name="c", subcore_axis_name="s")

    sweep = pl.kernel(
        functools.partial(_sweep_body, num_cores=info.num_cores),
        mesh=mesh,
        compiler_params=cparams,
        out_type=(jax.ShapeDtypeStruct((BATCH * EMBED_DIM,), jnp.float32),
                  jax.ShapeDtypeStruct((BATCH * EMBED_DIM,), jnp.float32)),
        scratch_types=[
            pltpu.VMEM((BATCH,), jnp.int32),
            pltpu.VMEM((LIST_CAP,), jnp.int32),
            pltpu.VMEM((LIST_CAP,), jnp.int32),
            pltpu.VMEM(((NGROUP + 1) * GROUP_CAP,), jnp.int32),
            pltpu.VMEM(((NGROUP + 1) * GROUP_CAP,), jnp.int32),
            pltpu.VMEM((4 * HIT_CAP,), jnp.int32),
            pltpu.VMEM((4 * HIT_CAP,), jnp.int32),
            pltpu.VMEM((8, 8, IDX_CHUNK), jnp.float32),
            pltpu.VMEM((8, 8, IDX_CHUNK), jnp.float32),
            pltpu.VMEM((8, 8, IDX_CHUNK), jnp.float32),
            pltpu.VMEM((8, 8, IDX_CHUNK), jnp.float32),
            pltpu.VMEM((OUT_CAP * EMBED_DIM,), jnp.float32),
            pltpu.SemaphoreType.DMA,
            pltpu.SemaphoreType.DMA,
            pltpu.SemaphoreType.DMA,
            pltpu.SemaphoreType.DMA,
            pltpu.SemaphoreType.DMA,
        ],
    )

    dot = pl.kernel(
        functools.partial(_dot_body, b_per_w=b_per_w,
                          num_cores=info.num_cores),
        mesh=mesh,
        compiler_params=cparams,
        out_type=jax.ShapeDtypeStruct((BATCH,), jnp.float32),
        scratch_types=[
            pltpu.VMEM((b_per_w,), jnp.int32),
            pltpu.VMEM((b_per_w,), jnp.int32),
            pltpu.VMEM((b_per_w * EMBED_DIM,), jnp.float32),
            pltpu.VMEM((b_per_w * EMBED_DIM,), jnp.float32),
            pltpu.VMEM((b_per_w,), jnp.float32),
            pltpu.VMEM((b_per_w,), jnp.float32),
            pltpu.VMEM((b_per_w,), jnp.float32),
            pltpu.VMEM((LANES,), jnp.float32),
            pltpu.SemaphoreType.DMA,
        ],
    )

    u32 = users.astype(jnp.int32)
    i32 = items.astype(jnp.int32)
    rows_u, rows_i = sweep(u32, i32,
                           user_embedding.T.reshape(8, 8, NROWS),
                           item_embedding.T.reshape(8, 8, NROWS))
    return dot(u32, i32, rows_u, rows_i,
               user_bias.reshape(user_bias.shape[0]),
               item_bias.reshape(item_bias.shape[0]),
               global_bias)


# speedup vs baseline: 35.9349x; 1.0021x over previous
import functools

import jax
import jax.numpy as jnp
from jax import lax
from jax.experimental import pallas as pl
from jax.experimental.pallas import tpu as pltpu
from jax.experimental.pallas import tpu_sc as plsc

BATCH = 16384
NROWS = 1000000
EMBED_DIM = 64
LANES = 16
IDX_CHUNK = 128
NSLAB = 7813
SLABS_PER_TILE = 244
LIST_CAP = 4112
NGROUP = 16
GROUP_CAP = 256
HIT_CAP = 528
OUT_CAP = 768


def _sweep_body(users_ref, items_ref, ue3_ref, ie3_ref, outu_ref, outi_ref,
                all_v, lval, lpos, gval, gpos, hitval, hitpos, slabA, slabB,
                slabC, slabD, outbuf, semA, semB, semC, semD, sem_out, *,
                num_cores):
    wid = lax.axis_index("s") * num_cores + lax.axis_index("c")
    lo = wid * SLABS_PER_TILE
    extra_s = jnp.where(wid < 5, NSLAB - 5 + wid, 0)

    lane = lax.iota(jnp.int32, LANES)
    chiv = [(m * LANES + lane) >> 3 for m in range(EMBED_DIM // LANES)]
    clov = [(m * LANES + lane) & 7 for m in range(EMBED_DIM // LANES)]

    def sweep_table(idx_ref, tab_ref, out1_ref):
        pltpu.sync_copy(idx_ref, all_v)

        def clr(ch, _):
            lval[pl.ds(ch * LANES, LANES)] = lax.broadcast(
                jnp.int32(-1), (LANES,))
            return 0
        lax.fori_loop(0, LIST_CAP // LANES, clr, 0)

        def clrg(ch, _):
            gval[pl.ds(ch * LANES, LANES)] = lax.broadcast(
                jnp.int32(-1), (LANES,))
            return 0
        lax.fori_loop(0, (NGROUP + 1) * GROUP_CAP // LANES, clrg, 0)

        def build(ch, cnt):
            sl = pl.ds(ch * LANES, LANES)
            vals = all_v[sl]
            slabv = vals >> 7
            m = ((slabv >= lo) & (slabv < lo + SLABS_PER_TILE)
                 | (slabv == extra_s))
            plsc.store_compressed(lval.at[pl.ds(cnt, LANES)], vals, mask=m)
            plsc.store_compressed(lpos.at[pl.ds(cnt, LANES)],
                                  ch * LANES + lane, mask=m)
            return cnt + plsc.all_reduce_population_count(m)[0]
        cnt = lax.fori_loop(0, BATCH // LANES, build, jnp.int32(0))
        nscan = (cnt + LANES - 1) >> 4

        gcnt_vec = jnp.zeros((LANES,), jnp.int32)
        extra_cnt = jnp.int32(0)
        for g in range(NGROUP + 1):
            def bing(ch, c2, g=g):
                sl = pl.ds(ch * LANES, LANES)
                vals = lval[sl]
                pv = lpos[sl]
                if g == NGROUP:
                    m = (vals >> 7) == extra_s
                else:
                    m = ((vals >> 7) - lo) >> 4 == g
                plsc.store_compressed(
                    gval.at[pl.ds(g * GROUP_CAP + c2, LANES)], vals, mask=m)
                plsc.store_compressed(
                    gpos.at[pl.ds(g * GROUP_CAP + c2, LANES)], pv, mask=m)
                return c2 + plsc.all_reduce_population_count(m)[0]
            cg = lax.fori_loop(0, nscan, bing, jnp.int32(0))
            if g == NGROUP:
                extra_cnt = cg
            else:
                gcnt_vec = jnp.where(lane == g, cg, gcnt_vec)

        def fire(buf, s, sem):
            start = pl.multiple_of(s * IDX_CHUNK, IDX_CHUNK)
            pltpu.async_copy(
                tab_ref.at[pl.ds(0, 4), :, pl.ds(start, IDX_CHUNK)],
                buf.at[pl.ds(0, 4)], sem)
            pltpu.async_copy(
                tab_ref.at[pl.ds(4, 4), :, pl.ds(start, IDX_CHUNK)],
                buf.at[pl.ds(4, 4)], sem)

        def wait(buf, sem):
            pltpu.make_async_copy(
                tab_ref.at[:, :, pl.ds(0, IDX_CHUNK)], buf, sem).wait()

        def scan_slab(s, slot, gbase, gn):
            hb0 = slot * HIT_CAP

            def scan(ch2, hcnt):
                sl = pl.ds(gbase + ch2 * LANES, LANES)
                vals = gval[sl]
                pv = gpos[sl]
                m = (vals >> 7) == s
                plsc.store_compressed(
                    hitval.at[pl.ds(hb0 + hcnt, LANES)], vals, mask=m)
                plsc.store_compressed(
                    hitpos.at[pl.ds(hb0 + hcnt, LANES)], pv, mask=m)
                return hcnt + plsc.all_reduce_population_count(m)[0]
            return lax.fori_loop(0, gn, scan, jnp.int32(0))

        def scan_group_of(s, slot):
            gid = (s - lo) >> 4
            gc = jnp.sum(jnp.where(lane == gid, gcnt_vec, 0))
            return scan_slab(s, slot, gid * GROUP_CAP, (gc + LANES - 1) >> 4)

        def extract_slab(buf, slot, hcnt, wcnt):
            hb0 = slot * HIT_CAP

            def extract(h, w):
                hb = hb0 + ((h >> 4) << 4)
                lsel = lane == (h & (LANES - 1))
                hv = hitval[pl.ds(hb, LANES)]
                val = jnp.sum(jnp.where(lsel, hv, 0))
                pv = hitpos[pl.ds(hb, LANES)]
                pos = jnp.sum(jnp.where(lsel, pv, 0))
                rm = lax.broadcast(val & (IDX_CHUNK - 1), (LANES,))
                for mch in range(EMBED_DIM // LANES):
                    g16 = plsc.load_gather(buf, [chiv[mch], clov[mch], rm])
                    outbuf[pl.ds(w * EMBED_DIM + mch * LANES, LANES)] = g16
                pltpu.async_copy(
                    outbuf.at[pl.ds(w * EMBED_DIM, EMBED_DIM)],
                    out1_ref.at[pl.ds(pos * EMBED_DIM, EMBED_DIM)], sem_out)
                return w + 1
            return lax.fori_loop(0, hcnt, extract, wcnt)

        slabs = [slabA, slabB, slabC, slabD]
        sems = [semA, semB, semC, semD]
        nquad = SLABS_PER_TILE // 4

        def cond_fire(slot, s, h):
            @pl.when(h > 0)
            def _():
                fire(slabs[slot], s, sems[slot])

        hpre = []
        for k in range(3):
            h = scan_group_of(lo + k, k)
            cond_fire(k, lo + k, h)
            hpre.append(h)

        def quad(qq, carry):
            wcnt, h0, h1, h2, h3 = carry
            hs = [h0, h1, h2, h3]
            s = lo + 4 * qq
            for k in range(4):
                slot_fill = 3 if k == 0 else k - 1
                hn = scan_group_of(s + 3 + k, slot_fill)
                cond_fire(slot_fill, s + 3 + k, hn)
                hk = hs[k]
                hs[slot_fill] = hn

                @pl.when(hk > 0)
                def _(k=k):
                    wait(slabs[k], sems[k])
                wcnt = extract_slab(slabs[k], k, hk, wcnt)
            return (wcnt, hs[0], hs[1], hs[2], hs[3])

        carry = lax.fori_loop(
            0, nquad, quad,
            (jnp.int32(0), hpre[0], hpre[1], hpre[2], jnp.int32(0)))
        wcnt = carry[0]

        he = scan_slab(extra_s, 0, NGROUP * GROUP_CAP,
                       (extra_cnt + LANES - 1) >> 4)
        cond_fire(0, extra_s, he)

        @pl.when(he > 0)
        def _():
            wait(slabA, semA)
        extract_slab(slabA, 0, he, wcnt)

        def drain(d, _):
            pltpu.make_async_copy(
                outbuf.at[pl.ds(0, EMBED_DIM)],
                out1_ref.at[pl.ds(0, EMBED_DIM)], sem_out).wait()
            return 0
        lax.fori_loop(0, cnt, drain, 0)

    sweep_table(users_ref, ue3_ref, outu_ref)
    sweep_table(items_ref, ie3_ref, outi_ref)


def _dot_body(users_ref, items_ref, outu_ref, outi_ref, ub_ref, ib_ref,
              gb_ref, out_ref, uidx_v, iidx_v, urows1, irows1,
              ub_v, ib_v, scores_v, gb_v, sem_bias, *, b_per_w, num_cores):
    wid = lax.axis_index("s") * num_cores + lax.axis_index("c")
    base = wid * b_per_w
    nchunk = b_per_w // IDX_CHUNK

    pltpu.sync_copy(users_ref.at[pl.ds(base, b_per_w)], uidx_v)
    pltpu.sync_copy(items_ref.at[pl.ds(base, b_per_w)], iidx_v)
    pltpu.sync_copy(gb_ref, gb_v.at[pl.ds(0, 1)])
    pltpu.sync_copy(outu_ref.at[pl.ds(base * EMBED_DIM, b_per_w * EMBED_DIM)],
                    urows1)
    pltpu.sync_copy(outi_ref.at[pl.ds(base * EMBED_DIM, b_per_w * EMBED_DIM)],
                    irows1)

    bias_handles = []
    for j in range(nchunk):
        sl = pl.ds(j * IDX_CHUNK, IDX_CHUNK)
        bias_handles.append(pltpu.async_copy(
            ub_ref.at[uidx_v.at[sl]], ub_v.at[sl], sem_bias))
        bias_handles.append(pltpu.async_copy(
            ib_ref.at[iidx_v.at[sl]], ib_v.at[sl], sem_bias))
    for h in bias_handles:
        h.wait()

    gb = gb_v[pl.ds(0, LANES)][0]
    lane = lax.iota(jnp.int32, LANES)

    def group(g, _):
        sl = pl.ds(g * LANES, LANES)
        score_vec = jnp.zeros((LANES,), jnp.float32)
        for j in range(LANES):
            e0 = (g * LANES + j) * EMBED_DIM
            acc = None
            for m in range(EMBED_DIM // LANES):
                u16 = urows1[pl.ds(e0 + m * LANES, LANES)]
                i16 = irows1[pl.ds(e0 + m * LANES, LANES)]
                p = u16 * i16
                acc = p if acc is None else acc + p
            s_val = jnp.sum(acc)
            score_vec = jnp.where(lane == j, s_val, score_vec)
        scores_v[sl] = score_vec + ub_v[sl] + ib_v[sl] + gb
        return 0

    lax.fori_loop(0, b_per_w // LANES, group, 0)

    pltpu.sync_copy(scores_v, out_ref.at[pl.ds(base, b_per_w)])


def kernel(users, items, user_embedding, item_embedding, user_bias,
           item_bias, global_bias):
    info = plsc.get_sparse_core_info()
    num_workers = info.num_cores * info.num_subcores
    b_per_w = BATCH // num_workers
    cparams = pltpu.CompilerParams(use_tc_tiling_on_sc=True,
                                   needs_layout_passes=False)
    mesh = plsc.VectorSubcoreMesh(core_axis_name="c", subcore_axis_name="s")

    sweep = pl.kernel(
        functools.partial(_sweep_body, num_cores=info.num_cores),
        mesh=mesh,
        compiler_params=cparams,
        out_type=(jax.ShapeDtypeStruct((BATCH * EMBED_DIM,), jnp.float32),
                  jax.ShapeDtypeStruct((BATCH * EMBED_DIM,), jnp.float32)),
        scratch_types=[
            pltpu.VMEM((BATCH,), jnp.int32),
            pltpu.VMEM((LIST_CAP,), jnp.int32),
            pltpu.VMEM((LIST_CAP,), jnp.int32),
            pltpu.VMEM(((NGROUP + 1) * GROUP_CAP,), jnp.int32),
            pltpu.VMEM(((NGROUP + 1) * GROUP_CAP,), jnp.int32),
            pltpu.VMEM((4 * HIT_CAP,), jnp.int32),
            pltpu.VMEM((4 * HIT_CAP,), jnp.int32),
            pltpu.VMEM((8, 8, IDX_CHUNK), jnp.float32),
            pltpu.VMEM((8, 8, IDX_CHUNK), jnp.float32),
            pltpu.VMEM((8, 8, IDX_CHUNK), jnp.float32),
            pltpu.VMEM((8, 8, IDX_CHUNK), jnp.float32),
            pltpu.VMEM((OUT_CAP * EMBED_DIM,), jnp.float32),
            pltpu.SemaphoreType.DMA,
            pltpu.SemaphoreType.DMA,
            pltpu.SemaphoreType.DMA,
            pltpu.SemaphoreType.DMA,
            pltpu.SemaphoreType.DMA,
        ],
    )

    dot = pl.kernel(
        functools.partial(_dot_body, b_per_w=b_per_w,
                          num_cores=info.num_cores),
        mesh=mesh,
        compiler_params=cparams,
        out_type=jax.ShapeDtypeStruct((BATCH,), jnp.float32),
        scratch_types=[
            pltpu.VMEM((b_per_w,), jnp.int32),
            pltpu.VMEM((b_per_w,), jnp.int32),
            pltpu.VMEM((b_per_w * EMBED_DIM,), jnp.float32),
            pltpu.VMEM((b_per_w * EMBED_DIM,), jnp.float32),
            pltpu.VMEM((b_per_w,), jnp.float32),
            pltpu.VMEM((b_per_w,), jnp.float32),
            pltpu.VMEM((b_per_w,), jnp.float32),
            pltpu.VMEM((LANES,), jnp.float32),
            pltpu.SemaphoreType.DMA,
        ],
    )

    u32 = users.astype(jnp.int32)
    i32 = items.astype(jnp.int32)
    rows_u, rows_i = sweep(u32, i32,
                           user_embedding.T.reshape(8, 8, NROWS),
                           item_embedding.T.reshape(8, 8, NROWS))
    return dot(u32, i32, rows_u, rows_i,
               user_bias.reshape(user_bias.shape[0]),
               item_bias.reshape(item_bias.shape[0]),
               global_bias)
